# baseline (speedup 1.0000x reference)
import jax
import jax.numpy as jnp
from jax import lax
from jax.experimental import pallas as pl
from jax.experimental.pallas import tpu as pltpu
from jax.experimental.pallas import tpu_sc as plsc

N = 10000
E = 320000
D = 128
T = 4
S = 16
R = 4 * N
RP = R // 8
CH = 80
NC = 2
NS = 16
NW = NC * NS
NCH = 125
EPT = NCH * CH
STRIPE = R // NS
NB = 8
LK = 4


def _sc_prologue(zeros_hbm, si_hbm, di_hbm, agg_sh, si_v, di_v):
    cid = lax.axis_index("c")
    sid = lax.axis_index("s")
    wid = sid * NC + cid
    pltpu.sync_copy(zeros_hbm, agg_sh.at[pl.ds(sid * STRIPE, STRIPE)])
    pltpu.sync_copy(si_hbm.at[wid], si_v)
    pltpu.sync_copy(di_hbm.at[wid], di_v)
    plsc.subcore_barrier()
    return cid, sid


def _sc_epilogue(out_hbm, agg_sh, cid, sid):
    plsc.subcore_barrier()
    pltpu.sync_copy(agg_sh.at[pl.ds(sid * STRIPE, STRIPE)],
                    out_hbm.at[cid].at[pl.ds(sid * STRIPE, STRIPE)])


def _sc_scatter_body(vals_hbm, si_hbm, di_hbm, zeros_hbm, out_hbm,
                     si_v, di_v, rows_v, gsem, ssem, table_sh, agg_sh):
    sid0 = lax.axis_index("s")
    pltpu.sync_copy(vals_hbm.at[pl.ds(sid0 * STRIPE, STRIPE)],
                    table_sh.at[pl.ds(sid0 * STRIPE, STRIPE)])
    cid, sid = _sc_prologue(zeros_hbm, si_hbm, di_hbm, agg_sh, si_v, di_v)
    gd = [None] * NCH
    sd = [None] * NCH
    for j in range(NCH + LK):
        if j < NCH:
            s = j % NB
            if j >= NB:
                sd[j - NB].wait()
            gd[j] = pltpu.async_copy(
                table_sh.at[si_v.at[j]], rows_v.at[s], gsem.at[s])
        if j >= LK:
            k = j - LK
            s = k % NB
            gd[k].wait()
            sd[k] = pltpu.async_copy(
                rows_v.at[s], agg_sh.at[di_v.at[k]], ssem.at[s], add=True)
    for k in range(NCH - NB, NCH):
        sd[k].wait()
    _sc_epilogue(out_hbm, agg_sh, cid, sid)


def _sc_deg_body(ones_hbm, si_hbm, di_hbm, zeros_hbm, out_hbm,
                 si_v, di_v, rows_v, ssem, agg_sh):
    cid, sid = _sc_prologue(zeros_hbm, si_hbm, di_hbm, agg_sh, si_v, di_v)
    pltpu.sync_copy(ones_hbm, rows_v)
    sd = [None] * NCH
    for j in range(NCH):
        s = j % NB
        if j >= NB:
            sd[j - NB].wait()
        sd[j] = pltpu.async_copy(
            rows_v, agg_sh.at[di_v.at[j]], ssem.at[s], add=True)
    for k in range(NCH - NB, NCH):
        sd[k].wait()
    _sc_epilogue(out_hbm, agg_sh, cid, sid)


_SC_OUT = jax.ShapeDtypeStruct((NC, R, S), jnp.float32)
_SC_PARAMS = pltpu.CompilerParams(use_tc_tiling_on_sc=False)


def _sc_mesh():
    return plsc.VectorSubcoreMesh(core_axis_name="c", subcore_axis_name="s")


def _sc_scatter(vals, si, di, zeros):
    fn = pl.kernel(
        _sc_scatter_body, mesh=_sc_mesh(), out_type=_SC_OUT,
        scratch_types=[
            pltpu.VMEM((NCH, CH), jnp.int32),
            pltpu.VMEM((NCH, CH), jnp.int32),
            pltpu.VMEM((NB, CH, S), jnp.float32),
            pltpu.SemaphoreType.DMA((NB,)),
            pltpu.SemaphoreType.DMA((NB,)),
            pltpu.VMEM_SHARED((R, S), jnp.float32),
            pltpu.VMEM_SHARED((R, S), jnp.float32),
        ],
        compiler_params=_SC_PARAMS,
    )
    return fn(vals, si, di, zeros)


def _sc_deg(ones_rows, si, di, zeros):
    fn = pl.kernel(
        _sc_deg_body, mesh=_sc_mesh(), out_type=_SC_OUT,
        scratch_types=[
            pltpu.VMEM((NCH, CH), jnp.int32),
            pltpu.VMEM((NCH, CH), jnp.int32),
            pltpu.VMEM((CH, S), jnp.float32),
            pltpu.SemaphoreType.DMA((NB,)),
            pltpu.VMEM_SHARED((R, S), jnp.float32),
        ],
        compiler_params=_SC_PARAMS,
    )
    return fn(ones_rows, si, di, zeros)


BR = 1000


def _spec(shape):
    if len(shape) == 3:
        return pl.BlockSpec((shape[0], BR, 128), lambda i: (0, i, 0))
    if shape[0] == RP:
        return pl.BlockSpec((BR, shape[1]), lambda i: (i, 0))
    return pl.BlockSpec(shape, lambda i: (0, 0))


def _ew_call(body, ins, n_out, out_minor=128):
    return pl.pallas_call(
        body,
        grid=(RP // BR,),
        in_specs=[_spec(a.shape) for a in ins],
        out_specs=[_spec((RP, out_minor))] * n_out,
        out_shape=[jax.ShapeDtypeStruct((RP, out_minor), jnp.float32)] * n_out,
    )(*ins)


def _mm_body(x_ref, w_ref, o_ref):
    xb = x_ref[...]
    for i in range(T):
        o_ref[:, i * S:(i + 1) * S] = jnp.dot(
            xb, w_ref[i], preferred_element_type=jnp.float32)


def _norm_body(degp_ref, h1_ref, dinv_ref, g_ref):
    deg = degp_ref[0] + degp_ref[1]
    dinv = lax.rsqrt(1.0 + deg)
    dinv_ref[...] = dinv
    g_ref[...] = h1_ref[...] * dinv


def _layer1_body(aggp_ref, g_ref, dinv_ref, b1_ref, r_ref, gr_ref):
    dinv = dinv_ref[...]
    agg = (aggp_ref[0] + aggp_ref[1]) * dinv
    r_ = jnp.maximum(agg + g_ref[...] * dinv + b1_ref[...], 0.0)
    r_ref[...] = r_
    gr_ref[...] = r_ * dinv


def _layer2_body(pp_ref, r_ref, dinv_ref, w2_ref, m_ref, b2_ref, o_ref):
    dinv = dinv_ref[...]
    z = (pp_ref[0] + pp_ref[1]) * dinv + r_ref[...] * dinv * dinv
    zw = z * w2_ref[...]
    o_ref[...] = jnp.dot(zw, m_ref[...],
                         preferred_element_type=jnp.float32) + b2_ref[...]


def kernel(x, edge_index, edge_attr, W1, b1, W2, b2):
    f32 = jnp.float32
    t = (edge_attr @ jnp.arange(T, dtype=f32)).astype(jnp.int32)
    si = (edge_index[0] * T + t).reshape(NW, NCH, CH)
    di = (edge_index[1] * T + t).reshape(NW, NCH, CH)
    zeros = jnp.zeros((STRIPE, S), f32)
    ones_rows = jnp.ones((CH, S), f32)

    y = pl.pallas_call(
        _mm_body,
        grid=(RP // BR,),
        in_specs=[pl.BlockSpec((2 * BR, D), lambda i: (i, 0)),
                  pl.BlockSpec((T, D, S), lambda i: (0, 0, 0))],
        out_specs=pl.BlockSpec((2 * BR, T * S), lambda i: (i, 0)),
        out_shape=jax.ShapeDtypeStruct((N, T * S), f32),
    )(x, W1)
    h1p = y.reshape(RP, 128)

    degp = _sc_deg(ones_rows, si, di, zeros)
    degp128 = degp.reshape(NC, RP, 128)

    dinvp, gp = _ew_call(_norm_body, [degp128, h1p], 2)

    aggp = _sc_scatter(gp.reshape(R, S), si, di, zeros)

    b1t = jnp.tile(b1.reshape(1, T * S), (1, 2))
    rp_, grp = _ew_call(_layer1_body, [aggp.reshape(NC, RP, 128), gp,
                                       dinvp, b1t], 2)

    pp = _sc_scatter(grp.reshape(R, S), si, di, zeros)

    w2t = jnp.tile(W2[:, :, 0].reshape(1, T * S), (1, 2))
    m = jnp.repeat(jnp.eye(8, dtype=f32), S, axis=0)
    b2t = jnp.tile(b2.reshape(1, T), (1, 2))
    (out,) = _ew_call(_layer2_body, [pp.reshape(NC, RP, 128), rp_,
                                     dinvp, w2t, m, b2t], 1, out_minor=8)
    return out.reshape(N, T)


# speedup vs baseline: 1.0549x; 1.0549x over previous
import jax
import jax.numpy as jnp
from jax import lax
from jax.experimental import pallas as pl
from jax.experimental.pallas import tpu as pltpu
from jax.experimental.pallas import tpu_sc as plsc

N = 10000
E = 320000
D = 128
T = 4
S = 16
R = 4 * N
RP = R // 8
CH = 80
NC = 2
NS = 16
NW = NC * NS
NCH = 125
EPT = NCH * CH
STRIPE = R // NS
NB = 8
LK = 4


def _sc_prologue(zeros_hbm, si_hbm, di_hbm, agg_sh, si_v, di_v):
    cid = lax.axis_index("c")
    sid = lax.axis_index("s")
    wid = sid * NC + cid
    pltpu.sync_copy(zeros_hbm.at[pl.ds(sid * STRIPE, STRIPE)],
                    agg_sh.at[pl.ds(sid * STRIPE, STRIPE)])
    pltpu.sync_copy(si_hbm.at[wid], si_v)
    pltpu.sync_copy(di_hbm.at[wid], di_v)
    plsc.subcore_barrier()
    return cid, sid


def _sc_epilogue(out_hbm, agg_sh, cid, sid):
    plsc.subcore_barrier()
    pltpu.sync_copy(agg_sh.at[pl.ds(sid * STRIPE, STRIPE)],
                    out_hbm.at[cid].at[pl.ds(sid * STRIPE, STRIPE)])


def _sc_scatter_body(vals_hbm, si_hbm, di_hbm, zeros_hbm, out_hbm,
                     si_v, di_v, rows_v, gsem, ssem, table_sh, agg_sh):
    sid0 = lax.axis_index("s")
    pltpu.sync_copy(vals_hbm.at[pl.ds(sid0 * STRIPE, STRIPE)],
                    table_sh.at[pl.ds(sid0 * STRIPE, STRIPE)])
    cid, sid = _sc_prologue(zeros_hbm, si_hbm, di_hbm, agg_sh, si_v, di_v)
    gd = [None] * NCH
    sd = [None] * NCH
    for j in range(NCH + LK):
        if j < NCH:
            s = j % NB
            if j >= NB:
                sd[j - NB].wait()
            gd[j] = pltpu.async_copy(
                table_sh.at[si_v.at[j]], rows_v.at[s], gsem.at[s])
        if j >= LK:
            k = j - LK
            s = k % NB
            gd[k].wait()
            sd[k] = pltpu.async_copy(
                rows_v.at[s], agg_sh.at[di_v.at[k]], ssem.at[s], add=True)
    for k in range(NCH - NB, NCH):
        sd[k].wait()
    _sc_epilogue(out_hbm, agg_sh, cid, sid)


def _sc_deg_body(ones_hbm, si_hbm, di_hbm, zeros_hbm, out_hbm,
                 si_v, di_v, rows_v, ssem, agg_sh):
    cid, sid = _sc_prologue(zeros_hbm, si_hbm, di_hbm, agg_sh, si_v, di_v)
    pltpu.sync_copy(ones_hbm, rows_v)
    sd = [None] * NCH
    for j in range(NCH):
        s = j % NB
        if j >= NB:
            sd[j - NB].wait()
        sd[j] = pltpu.async_copy(
            rows_v, agg_sh.at[di_v.at[j]], ssem.at[s], add=True)
    for k in range(NCH - NB, NCH):
        sd[k].wait()
    _sc_epilogue(out_hbm, agg_sh, cid, sid)


_SC_OUT = jax.ShapeDtypeStruct((NC, R, S), jnp.float32)
_SC_PARAMS = pltpu.CompilerParams(use_tc_tiling_on_sc=False)


def _sc_mesh():
    return plsc.VectorSubcoreMesh(core_axis_name="c", subcore_axis_name="s")


def _sc_scatter(vals, si, di, zeros):
    fn = pl.kernel(
        _sc_scatter_body, mesh=_sc_mesh(), out_type=_SC_OUT,
        scratch_types=[
            pltpu.VMEM((NCH, CH), jnp.int32),
            pltpu.VMEM((NCH, CH), jnp.int32),
            pltpu.VMEM((NB, CH, S), jnp.float32),
            pltpu.SemaphoreType.DMA((NB,)),
            pltpu.SemaphoreType.DMA((NB,)),
            pltpu.VMEM_SHARED((R, S), jnp.float32),
            pltpu.VMEM_SHARED((R, S), jnp.float32),
        ],
        compiler_params=_SC_PARAMS,
    )
    return fn(vals, si, di, zeros)


def _sc_deg(ones_rows, si, di, zeros):
    fn = pl.kernel(
        _sc_deg_body, mesh=_sc_mesh(), out_type=_SC_OUT,
        scratch_types=[
            pltpu.VMEM((NCH, CH), jnp.int32),
            pltpu.VMEM((NCH, CH), jnp.int32),
            pltpu.VMEM((CH, S), jnp.float32),
            pltpu.SemaphoreType.DMA((NB,)),
            pltpu.VMEM_SHARED((R, S), jnp.float32),
        ],
        compiler_params=_SC_PARAMS,
    )
    return fn(ones_rows, si, di, zeros)


BR = 1000


def _spec(shape):
    if len(shape) == 3:
        return pl.BlockSpec((shape[0], BR, 128), lambda i: (0, i, 0))
    if shape[0] == RP:
        return pl.BlockSpec((BR, shape[1]), lambda i: (i, 0))
    return pl.BlockSpec(shape, lambda i: (0, 0))


def _ew_call(body, ins, n_out, out_minor=128):
    return pl.pallas_call(
        body,
        grid=(RP // BR,),
        in_specs=[_spec(a.shape) for a in ins],
        out_specs=[_spec((RP, out_minor))] * n_out,
        out_shape=[jax.ShapeDtypeStruct((RP, out_minor), jnp.float32)] * n_out,
    )(*ins)


def _mm_body(x_ref, w_ref, o_ref):
    xb = x_ref[...]
    for i in range(T):
        o_ref[:, i * S:(i + 1) * S] = jnp.dot(
            xb, w_ref[i], preferred_element_type=jnp.float32)


def _norm_body(degp_ref, h1_ref, dinv_ref, g_ref):
    deg = degp_ref[0] + degp_ref[1]
    dinv = lax.rsqrt(1.0 + deg)
    dinv_ref[...] = dinv
    g_ref[...] = h1_ref[...] * dinv


def _layer1_body(aggp_ref, g_ref, dinv_ref, b1_ref, r_ref, gr_ref):
    dinv = dinv_ref[...]
    agg = (aggp_ref[0] + aggp_ref[1]) * dinv
    r_ = jnp.maximum(agg + g_ref[...] * dinv + b1_ref[...], 0.0)
    r_ref[...] = r_
    gr_ref[...] = r_ * dinv


def _layer2_body(pp_ref, r_ref, dinv_ref, w2_ref, m_ref, b2_ref, o_ref):
    dinv = dinv_ref[...]
    z = (pp_ref[0] + pp_ref[1]) * dinv + r_ref[...] * dinv * dinv
    zw = z * w2_ref[...]
    o_ref[...] = jnp.dot(zw, m_ref[...],
                         preferred_element_type=jnp.float32) + b2_ref[...]


def kernel(x, edge_index, edge_attr, W1, b1, W2, b2):
    f32 = jnp.float32
    t = (edge_attr @ jnp.arange(T, dtype=f32)).astype(jnp.int32)
    si = (edge_index[0] * T + t).reshape(NW, NCH, CH)
    di = (edge_index[1] * T + t).reshape(NW, NCH, CH)
    zeros = jnp.zeros((R, S), f32)
    ones_rows = jnp.ones((CH, S), f32)

    y = pl.pallas_call(
        _mm_body,
        grid=(RP // BR,),
        in_specs=[pl.BlockSpec((2 * BR, D), lambda i: (i, 0)),
                  pl.BlockSpec((T, D, S), lambda i: (0, 0, 0))],
        out_specs=pl.BlockSpec((2 * BR, T * S), lambda i: (i, 0)),
        out_shape=jax.ShapeDtypeStruct((N, T * S), f32),
    )(x, W1)
    h1p = y.reshape(RP, 128)

    degp = _sc_deg(ones_rows, si, di, zeros)
    degp128 = degp.reshape(NC, RP, 128)

    dinvp, gp = _ew_call(_norm_body, [degp128, h1p], 2)

    aggp = _sc_scatter(gp.reshape(R, S), si, di, zeros)

    b1t = jnp.tile(b1.reshape(1, T * S), (1, 2))
    rp_, grp = _ew_call(_layer1_body, [aggp.reshape(NC, RP, 128), gp,
                                       dinvp, b1t], 2)

    pp = _sc_scatter(grp.reshape(R, S), si, di, zeros)

    w2t = jnp.tile(W2[:, :, 0].reshape(1, T * S), (1, 2))
    m = jnp.repeat(jnp.eye(8, dtype=f32), S, axis=0)
    b2t = jnp.tile(b2.reshape(1, T), (1, 2))
    (out,) = _ew_call(_layer2_body, [pp.reshape(NC, RP, 128), rp_,
                                     dinvp, w2t, m, b2t], 1, out_minor=8)
    return out.reshape(N, T)
